# Initial kernel scaffold; baseline (speedup 1.0000x reference)
import jax
import jax.numpy as jnp
from jax.experimental import pallas as pl


def kernel(x, edge_index, additional_x, W_rel1, W_root1, b1, W_rel2, W_root2, b2, W_rel3, W_root3, b3):
    raise NotImplementedError("write your pallas kernel here")


# speedup vs baseline: 2.7406x; 2.7406x over previous
import functools

import jax
import jax.numpy as jnp
from jax import lax
from jax.experimental import pallas as pl
from jax.experimental.pallas import tpu as pltpu
from jax.experimental.pallas import tpu_sc as plsc

N = 10000
D = 128
E = 320000
NW = 32
EPW = 10240
EPAD = NW * EPW
CH = 128
K = EPW // CH
A = 10240
ZR = A // 16

_mesh = plsc.VectorSubcoreMesh(core_axis_name="c", subcore_axis_name="s")


@functools.partial(
    pl.kernel,
    out_type=jax.ShapeDtypeStruct((2, A, D), jnp.float32),
    mesh=_mesh,
    scratch_types=[
        pltpu.VMEM((K, CH), jnp.int32),
        pltpu.VMEM((K, CH), jnp.int32),
        pltpu.VMEM((CH, D), jnp.float32),
        pltpu.VMEM_SHARED((A, D), jnp.float32),
        pltpu.SemaphoreType.DMA,
    ],
)
def _sc_segsum(p_hbm, src_hbm, dst_hbm, zeros_hbm, out_hbm,
               src_v, dst_v, rows_v, acc_sh, sem):
    c = lax.axis_index("c")
    s = lax.axis_index("s")
    wid = s * 2 + c
    pltpu.sync_copy(zeros_hbm, acc_sh.at[pl.ds(s * ZR, ZR)])
    pltpu.sync_copy(src_hbm.at[pl.ds(wid * K, K), :], src_v)
    pltpu.sync_copy(dst_hbm.at[pl.ds(wid * K, K), :], dst_v)
    plsc.subcore_barrier()

    def body(j, carry):
        pltpu.async_copy(p_hbm.at[src_v.at[j]], rows_v, sem).wait()
        pltpu.sync_copy(rows_v, acc_sh.at[dst_v.at[j]], add=True)
        return carry

    lax.fori_loop(0, K, body, 0)
    plsc.subcore_barrier()
    pltpu.sync_copy(acc_sh.at[pl.ds(s * ZR, ZR)],
                    out_hbm.at[c, pl.ds(s * ZR, ZR)])


_R = 400


def _tc_stage1_body(h_ref, wr_ref, wo_ref, b_ref, p_ref, root_ref):
    h = h_ref[...]
    p_ref[...] = jnp.dot(h, wr_ref[...], preferred_element_type=jnp.float32)
    root_ref[...] = (jnp.dot(h, wo_ref[...], preferred_element_type=jnp.float32)
                     + b_ref[...])


def _tc_mid_body(a0_ref, a1_ref, r_ref, wr_ref, wo_ref, b_ref, p_ref, root_ref):
    h = jnp.maximum(a0_ref[...] + a1_ref[...] + r_ref[...], 0.0)
    p_ref[...] = jnp.dot(h, wr_ref[...], preferred_element_type=jnp.float32)
    root_ref[...] = (jnp.dot(h, wo_ref[...], preferred_element_type=jnp.float32)
                     + b_ref[...])


def _tc_final_body(a0_ref, a1_ref, r_ref, ax_ref, o_ref):
    o_ref[...] = a0_ref[...] + a1_ref[...] + r_ref[...] + ax_ref[...]


def _rows_spec(din):
    return pl.BlockSpec((_R, din), lambda i: (i, 0))


def _full_spec(din):
    return pl.BlockSpec((din, D), lambda i: (0, 0))


_B_SPEC = pl.BlockSpec((1, D), lambda i: (0, 0))


def _tc_stage1(h, wr, wo, b):
    return pl.pallas_call(
        _tc_stage1_body,
        grid=(N // _R,),
        in_specs=[_rows_spec(h.shape[1]), _full_spec(h.shape[1]),
                  _full_spec(h.shape[1]), _B_SPEC],
        out_specs=[_rows_spec(D), _rows_spec(D)],
        out_shape=[jax.ShapeDtypeStruct((N, D), jnp.float32),
                   jax.ShapeDtypeStruct((N, D), jnp.float32)],
    )(h, wr, wo, b)


def _tc_mid(a0, a1, r, wr, wo, b):
    return pl.pallas_call(
        _tc_mid_body,
        grid=(N // _R,),
        in_specs=[_rows_spec(D), _rows_spec(D), _rows_spec(D),
                  _full_spec(D), _full_spec(D), _B_SPEC],
        out_specs=[_rows_spec(D), _rows_spec(D)],
        out_shape=[jax.ShapeDtypeStruct((N, D), jnp.float32),
                   jax.ShapeDtypeStruct((N, D), jnp.float32)],
    )(a0, a1, r, wr, wo, b)


def _tc_final(a0, a1, r, ax):
    return pl.pallas_call(
        _tc_final_body,
        grid=(N // _R,),
        in_specs=[_rows_spec(D), _rows_spec(D), _rows_spec(D), _rows_spec(D)],
        out_specs=_rows_spec(D),
        out_shape=jax.ShapeDtypeStruct((N, D), jnp.float32),
    )(a0, a1, r, ax)


def kernel(x, edge_index, additional_x, W_rel1, W_root1, b1,
           W_rel2, W_root2, b2, W_rel3, W_root3, b3):
    h0 = jnp.concatenate([x, additional_x], axis=1)
    src = edge_index[0]
    dst = edge_index[1]
    pad = EPAD - E
    src_p = jnp.concatenate([src, jnp.zeros((pad,), jnp.int32)]).reshape(NW * K, CH)
    dst_p = jnp.concatenate([dst, jnp.full((pad,), N, jnp.int32)]).reshape(NW * K, CH)
    zeros = jnp.zeros((ZR, D), jnp.float32)

    p1, root1 = _tc_stage1(h0, W_rel1, W_root1, b1.reshape(1, D))
    acc = _sc_segsum(p1, src_p, dst_p, zeros)
    p2, root2 = _tc_mid(acc[0, :N], acc[1, :N], root1, W_rel2, W_root2,
                        b2.reshape(1, D))
    acc = _sc_segsum(p2, src_p, dst_p, zeros)
    p3, root3 = _tc_mid(acc[0, :N], acc[1, :N], root2, W_rel3, W_root3,
                        b3.reshape(1, D))
    acc = _sc_segsum(p3, src_p, dst_p, zeros)
    return _tc_final(acc[0, :N], acc[1, :N], root3, additional_x)
